# Initial kernel scaffold; baseline (speedup 1.0000x reference)
import jax
import jax.numpy as jnp
from jax.experimental import pallas as pl


def kernel(embeddings, pids, lut, cq):
    raise NotImplementedError("write your pallas kernel here")


# speedup vs baseline: 5.7193x; 5.7193x over previous
import functools

import jax
import jax.numpy as jnp
from jax import lax
from jax.experimental import pallas as pl
from jax.experimental.pallas import tpu as pltpu
from jax.experimental.pallas import tpu_sc as plsc

N_LUT = 100000
N_CQ = 5000
DIM_ = 256
B_ = 1024
MOM = 0.5
SCAL = 30.0
LOG2E = 1.4426950408889634
LN2 = 0.6931471805599453

R_ = 5120
NBLK = (N_LUT + R_ - 1) // R_
LUT_LAST = N_LUT - (NBLK - 1) * R_
CQBLK = (N_CQ + R_ - 1) // R_
CQ_LAST = N_CQ - (CQBLK - 1) * R_
GRID = NBLK + CQBLK


def _mm_body(emb_ref, lut_ref, cq_ref, g_ref, out_ref, upd_ref,
             loss_ref, m_ref, s_ref, ebf_ref, cp_sem):
    i = pl.program_id(0)

    @pl.when(i == 0)
    def _prep():
        ebf_ref[...] = (emb_ref[...] * (SCAL * LOG2E)).astype(jnp.bfloat16)

    def _accum(blk, valid, first=False):
        logits = lax.dot_general(
            ebf_ref[...], blk.astype(jnp.bfloat16), (((1,), (1,)), ((), ())),
            preferred_element_type=jnp.float32)
        if valid is not None:
            col = lax.broadcasted_iota(jnp.int32, (B_, R_), 1)
            logits = jnp.where(col < valid, logits, -jnp.inf)
        if first:
            m_ref[...] = jnp.max(logits, axis=1, keepdims=True) + 40.0
            s_ref[...] = jnp.zeros((B_, 1), jnp.float32)
        s_ref[...] += jnp.sum(jnp.exp2(logits - m_ref[...]), axis=1,
                              keepdims=True)

    copy_dma = pltpu.make_async_copy(lut_ref, out_ref, cp_sem)

    @pl.when(i < NBLK)
    def _copy_start():
        copy_dma.start()

    @pl.when(i == 0)
    def _first_block():
        _accum(lut_ref[...], None, first=True)

    @pl.when(jnp.logical_and(i > 0, i < NBLK - 1))
    def _lut_full():
        _accum(lut_ref[...], None)

    @pl.when(i == NBLK - 1)
    def _lut_partial():
        _accum(lut_ref[...], LUT_LAST)

    @pl.when(jnp.logical_and(i >= NBLK, i < GRID - 1))
    def _cq_full():
        _accum(cq_ref[...], None)

    @pl.when(i == GRID - 1)
    def _cq_partial():
        _accum(cq_ref[...], CQ_LAST)

    @pl.when(i == GRID - 1)
    def _finalize():
        g = g_ref[...]
        emb = emb_ref[...]
        lse = LN2 * m_ref[...] + jnp.log(s_ref[...])
        rowdot = jnp.sum(emb * g, axis=1, keepdims=True)
        loss_ref[0, 0] = jnp.sum(lse - SCAL * rowdot) / B_
        u = MOM * g + (1.0 - MOM) * emb
        norm = jnp.sqrt(jnp.sum(u * u, axis=1, keepdims=True))
        upd_ref[...] = u / jnp.maximum(norm, 1e-12)

    @pl.when(i < NBLK)
    def _copy_drain():
        copy_dma.wait()


_mm_call = pl.pallas_call(
    _mm_body,
    grid=(GRID,),
    in_specs=[
        pl.BlockSpec((B_, DIM_), lambda i: (0, 0)),
        pl.BlockSpec((R_, DIM_), lambda i: (jnp.minimum(i, NBLK - 1), 0)),
        pl.BlockSpec((R_, DIM_),
                     lambda i: (jnp.clip(i - NBLK, 0, CQBLK - 1), 0)),
        pl.BlockSpec((B_, DIM_), lambda i: (0, 0)),
    ],
    out_specs=[
        pl.BlockSpec((R_, DIM_), lambda i: (jnp.minimum(i, NBLK - 1), 0)),
        pl.BlockSpec((B_, DIM_), lambda i: (0, 0)),
        pl.BlockSpec(memory_space=pltpu.SMEM),
    ],
    out_shape=[
        jax.ShapeDtypeStruct((N_LUT, DIM_), jnp.float32),
        jax.ShapeDtypeStruct((B_, DIM_), jnp.float32),
        jax.ShapeDtypeStruct((1, 1), jnp.float32),
    ],
    scratch_shapes=[
        pltpu.VMEM((B_, 1), jnp.float32),
        pltpu.VMEM((B_, 1), jnp.float32),
        pltpu.VMEM((B_, DIM_), jnp.bfloat16),
        pltpu.SemaphoreType.DMA,
    ],
    compiler_params=pltpu.CompilerParams(
        dimension_semantics=("arbitrary",),
        vmem_limit_bytes=110 * 1024 * 1024,
    ),
)


NCORES = 2
NSUB = 16
NW = NCORES * NSUB
BPW = B_ // NW


@functools.cache
def _sc_kernels():
    mesh = plsc.VectorSubcoreMesh(
        core_axis_name="c", subcore_axis_name="s",
        num_cores=NCORES, num_subcores=NSUB)

    @functools.partial(
        pl.kernel,
        out_type=jax.ShapeDtypeStruct((B_, DIM_), jnp.float32),
        mesh=mesh,
        scratch_types=[
            pltpu.VMEM((BPW,), jnp.int32),
            pltpu.VMEM((BPW, DIM_), jnp.float32),
            pltpu.SemaphoreType.DMA,
        ],
    )
    def sc_gather(lut_hbm, idx_hbm, out_hbm, idx_v, rows_v, sem):
        wid = lax.axis_index("s") * NCORES + lax.axis_index("c")
        base = wid * BPW
        pltpu.sync_copy(idx_hbm.at[pl.ds(base, BPW)], idx_v)
        pltpu.async_copy(lut_hbm.at[idx_v], rows_v, sem).wait()
        pltpu.sync_copy(rows_v, out_hbm.at[pl.ds(base, BPW)])

    @functools.partial(
        pl.kernel,
        out_type=(),
        mesh=mesh,
        scratch_types=[
            pltpu.VMEM((BPW,), jnp.int32),
            pltpu.VMEM((BPW,), jnp.int32),
            pltpu.VMEM((BPW, DIM_), jnp.float32),
            pltpu.SemaphoreType.DMA,
        ],
    )
    def sc_scatter(upd_hbm, wsrc_hbm, pid_hbm, newlut_ref,
                   widx_v, pidx_v, rows_v, sem):
        wid = lax.axis_index("s") * NCORES + lax.axis_index("c")
        base = wid * BPW
        pltpu.sync_copy(wsrc_hbm.at[pl.ds(base, BPW)], widx_v)
        pltpu.sync_copy(pid_hbm.at[pl.ds(base, BPW)], pidx_v)
        pltpu.async_copy(upd_hbm.at[widx_v], rows_v, sem).wait()
        pltpu.async_copy(rows_v, newlut_ref.at[pidx_v], sem).wait()

    return sc_gather, sc_scatter


def _winner_permutation(pids):
    order = jnp.argsort(pids, stable=True)
    sp = pids[order]
    is_last = jnp.concatenate(
        [sp[1:] != sp[:-1], jnp.ones((1,), dtype=bool)])
    pos = jnp.where(is_last, jnp.arange(B_, dtype=jnp.int32), B_)
    wpos = jnp.flip(lax.cummin(jnp.flip(pos)))
    winner_sorted = order[wpos].astype(jnp.int32)
    return jnp.zeros((B_,), jnp.int32).at[order].set(winner_sorted)


def kernel(embeddings, pids, lut, cq):
    sc_gather, sc_scatter = _sc_kernels()
    pids = pids.astype(jnp.int32)
    winner_src = _winner_permutation(pids)
    gathered = sc_gather(lut, pids)
    new_lut0, updated, loss = _mm_call(embeddings, lut, cq, gathered)
    nl_ref = jax.new_ref(new_lut0)
    sc_scatter(updated, winner_src, pids, nl_ref)
    return loss[0, 0], jax.freeze(nl_ref)
